# baseline (speedup 1.0000x reference)
import functools

import jax
import jax.numpy as jnp
from jax import lax
from jax.experimental import pallas as pl
from jax.experimental.pallas import tpu as pltpu

B, NI, NT, H, D = 16, 1024, 77, 768, 64
NTP = 80
NUM_QUERY = 16
BS = 2
_SQRT_HALF = 0.7071067811865476


def _gelu(x):
    return 0.5 * x * (1.0 + lax.erf(x * _SQRT_HALF))


def _ln(x, scale, bias):
    m = x.mean(-1, keepdims=True)
    v = ((x - m) ** 2).mean(-1, keepdims=True)
    inv = 1.0 / jnp.sqrt(v + 1e-5)
    return (x - m) * inv * scale + bias


def _select_body(img_ref, txt_ref, gum_ref,
                 ln_t_scale, ln_t_bias, W_t, b_t,
                 ln_i_scale, ln_i_bias, W_i, b_i,
                 W_tw1, b_tw1, W_tw2, b_tw2,
                 temp_ref, lam_ref, sel_ref):
  for j in range(BS):
    t = txt_ref[j]
    x = img_ref[j]
    g = gum_ref[j]

    tc = _gelu(_ln(t, ln_t_scale[0], ln_t_bias[0]) @ W_t[...] + b_t[0]) + t[:, :D]
    tw = _gelu(t @ W_tw1[...] + b_tw1[0]) @ W_tw2[...] + b_tw2[0]
    sg = jax.nn.sigmoid(tw)
    sg_m = sg - jnp.max(sg, axis=0, keepdims=True)
    e_t = jnp.exp(sg_m)
    w = e_t / jnp.sum(e_t, axis=0, keepdims=True)
    wt = tc * w
    wt = jnp.concatenate([wt, jnp.zeros((NTP - NT, D), jnp.float32)], axis=0)

    im = _gelu(_ln(x, ln_i_scale[0], ln_i_bias[0]) @ W_i[...] + b_i[0]) + x[:, :D]

    logits = lax.dot_general(wt, im, (((1,), (1,)), ((), ())))
    logits = logits / (jnp.abs(temp_ref[0, 0]) + 1e-6)

    row = lax.broadcasted_iota(jnp.int32, (NTP, 1), 0)
    mx = jnp.max(logits, axis=1, keepdims=True)
    e = jnp.where(row < NT, jnp.exp(logits - mx), 0.0)
    s = jnp.sum(e, axis=1, keepdims=True)
    s = jnp.where(row < NT, s, 1.0)
    sm = e / s
    scores = jnp.max(sm, axis=0, keepdims=True)

    ssum = jnp.sum(scores)
    lp = jnp.log(scores / (ssum + 1e-6))
    ls = jnp.log(scores)
    lam = lam_ref[0, 0]

    lin = lax.broadcasted_iota(jnp.int32, (1, NI), 1)
    lane16 = lax.broadcasted_iota(jnp.int32, (1, NUM_QUERY), 1)
    ones_row = jnp.ones((1, NTP), jnp.float32)

    def argmax_first(v):
        m = jnp.max(v)
        return jnp.min(jnp.where(v == m, lin, NI))

    idx = argmax_first(lp + g)
    sel_vec = jnp.full((1, NUM_QUERY), idx, jnp.int32)
    M = jnp.where(lin == idx, -jnp.inf, jnp.zeros((1, NI), jnp.float32))
    S = jnp.zeros((1, NI), jnp.float32)

    for k in range(1, NUM_QUERY):
        oh = (lin == idx).astype(jnp.float32)
        rrow = lax.dot_general(oh, logits, (((1,), (1,)), ((), ())))
        r = rrow.reshape(NTP, 1)
        diff2 = (logits - r) ** 2
        d2 = lax.dot_general(ones_row, diff2, (((1,), (0,)), ((), ())))
        S = S + jnp.sqrt(d2)
        comb = ls + lam * (S / float(k)) + M
        idx = argmax_first(comb)
        sel_vec = jnp.where(lane16 == k, idx, sel_vec)
        M = jnp.where(lin == idx, -jnp.inf, M)

    sel_ref[j] = sel_vec


@functools.partial(jax.jit, static_argnames=())
def _run(image_features, text_features, gumbel,
         ln_t_scale, ln_t_bias, W_t, b_t, ln_i_scale, ln_i_bias, W_i, b_i,
         W_tw1, b_tw1, W_tw2, b_tw2, temperature, diversity_lambda):
    full = lambda shape: pl.BlockSpec(shape, lambda b: (0,) * len(shape))
    grid_spec = pl.GridSpec(
        grid=(B // BS,),
        in_specs=[
            pl.BlockSpec((BS, NI, H), lambda b: (b, 0, 0)),
            pl.BlockSpec((BS, NT, H), lambda b: (b, 0, 0)),
            pl.BlockSpec((BS, 1, NI), lambda b: (b, 0, 0)),
            full((1, H)), full((1, H)), full((H, D)), full((1, D)),
            full((1, H)), full((1, H)), full((H, D)), full((1, D)),
            full((H, D)), full((1, D)), full((D, 1)), full((1, 1)),
            pl.BlockSpec(memory_space=pltpu.SMEM),
            pl.BlockSpec(memory_space=pltpu.SMEM),
        ],
        out_specs=pl.BlockSpec((BS, 1, NUM_QUERY), lambda b: (b, 0, 0)),
    )
    return pl.pallas_call(
        _select_body,
        grid_spec=grid_spec,
        out_shape=jax.ShapeDtypeStruct((B, 1, NUM_QUERY), jnp.int32),
    )(image_features, text_features, gumbel,
      ln_t_scale.reshape(1, H), ln_t_bias.reshape(1, H), W_t, b_t.reshape(1, D),
      ln_i_scale.reshape(1, H), ln_i_bias.reshape(1, H), W_i, b_i.reshape(1, D),
      W_tw1, b_tw1.reshape(1, D), W_tw2, b_tw2.reshape(1, 1),
      temperature.reshape(1, 1), diversity_lambda.reshape(1, 1))


def kernel(image_features, text_features, ln_t_scale, ln_t_bias, W_t, b_t,
           ln_i_scale, ln_i_bias, W_i, b_i, W_tw1, b_tw1, W_tw2, b_tw2,
           temperature, diversity_lambda):
    gum = jax.random.gumbel(jax.random.key(42), (B, NI), jnp.float32)
    sel = _run(image_features, text_features, gum.reshape(B, 1, NI),
               ln_t_scale, ln_t_bias, W_t, b_t, ln_i_scale, ln_i_bias, W_i,
               b_i, W_tw1, b_tw1, W_tw2, b_tw2,
               jnp.asarray(temperature, jnp.float32),
               jnp.asarray(diversity_lambda, jnp.float32))
    return sel.reshape(B, NUM_QUERY).astype(jnp.int64)


# speedup vs baseline: 1.1813x; 1.1813x over previous
import functools

import jax
import jax.numpy as jnp
from jax import lax
from jax.experimental import pallas as pl
from jax.experimental.pallas import tpu as pltpu

B, NI, NT, H, D = 16, 1024, 77, 768, 64
NTP = 80
NUM_QUERY = 16
BS = 2
_SQRT_HALF = 0.7071067811865476


def _gelu(x):
    return 0.5 * x * (1.0 + lax.erf(x * _SQRT_HALF))


def _ln(x, scale, bias):
    m = x.mean(-1, keepdims=True)
    v = ((x - m) ** 2).mean(-1, keepdims=True)
    inv = 1.0 / jnp.sqrt(v + 1e-5)
    return (x - m) * inv * scale + bias


def _select_body(img_ref, txt_ref, gum_ref,
                 ln_t_scale, ln_t_bias, W_t, b_t,
                 ln_i_scale, ln_i_bias, W_i, b_i,
                 W_tw1, b_tw1, W_tw2, b_tw2,
                 temp_ref, lam_ref, sel_ref):
  for j in range(BS):
    t = txt_ref[j]
    x = img_ref[j]
    g = gum_ref[j]

    tc = _gelu(_ln(t, ln_t_scale[0], ln_t_bias[0]) @ W_t[...] + b_t[0]) + t[:, :D]
    tw = _gelu(t @ W_tw1[...] + b_tw1[0]) @ W_tw2[...] + b_tw2[0]
    sg = jax.nn.sigmoid(tw)
    sg_m = sg - jnp.max(sg, axis=0, keepdims=True)
    e_t = jnp.exp(sg_m)
    w = e_t / jnp.sum(e_t, axis=0, keepdims=True)
    wt = tc * w
    wt = jnp.concatenate([wt, jnp.zeros((NTP - NT, D), jnp.float32)], axis=0)

    im = _gelu(_ln(x, ln_i_scale[0], ln_i_bias[0]) @ W_i[...] + b_i[0]) + x[:, :D]

    logits = lax.dot_general(wt, im, (((1,), (1,)), ((), ())))
    logits = logits / (jnp.abs(temp_ref[0, 0]) + 1e-6)

    row = lax.broadcasted_iota(jnp.int32, (NTP, 1), 0)
    mx = jnp.max(logits, axis=1, keepdims=True)
    e = jnp.where(row < NT, jnp.exp(logits - mx), 0.0)
    s = jnp.sum(e, axis=1, keepdims=True)
    s = jnp.where(row < NT, s, 1.0)
    sm = e / s
    scores = jnp.max(sm, axis=0, keepdims=True)

    ssum = jnp.sum(scores)
    lp = jnp.log(scores / (ssum + 1e-6))
    ls = jnp.log(scores)
    lam = lam_ref[0, 0]

    lin = lax.broadcasted_iota(jnp.int32, (1, NI), 1)
    lane16 = lax.broadcasted_iota(jnp.int32, (1, NUM_QUERY), 1)
    ones_row = jnp.ones((1, NTP), jnp.float32)

    def argmax_first(v):
        m = jnp.max(v)
        return jnp.min(jnp.where(v == m, lin, NI))

    idx = argmax_first(lp + g)
    sel_vec = jnp.full((1, NUM_QUERY), idx, jnp.int32)
    M = jnp.where(lin == idx, -jnp.inf, jnp.zeros((1, NI), jnp.float32))
    S = jnp.zeros((1, NI), jnp.float32)

    for k in range(1, NUM_QUERY):
        masked = jnp.where(lin == idx, logits, 0.0)
        r = jnp.sum(masked, axis=1, keepdims=True)
        diff2 = (logits - r) ** 2
        d2 = lax.dot_general(ones_row, diff2, (((1,), (0,)), ((), ())))
        S = S + jnp.sqrt(d2)
        comb = ls + lam * (S / float(k)) + M
        idx = argmax_first(comb)
        sel_vec = jnp.where(lane16 == k, idx, sel_vec)
        M = jnp.where(lin == idx, -jnp.inf, M)

    sel_ref[j] = sel_vec


@functools.partial(jax.jit, static_argnames=())
def _run(image_features, text_features, gumbel,
         ln_t_scale, ln_t_bias, W_t, b_t, ln_i_scale, ln_i_bias, W_i, b_i,
         W_tw1, b_tw1, W_tw2, b_tw2, temperature, diversity_lambda):
    full = lambda shape: pl.BlockSpec(shape, lambda b: (0,) * len(shape))
    grid_spec = pl.GridSpec(
        grid=(B // BS,),
        in_specs=[
            pl.BlockSpec((BS, NI, H), lambda b: (b, 0, 0)),
            pl.BlockSpec((BS, NT, H), lambda b: (b, 0, 0)),
            pl.BlockSpec((BS, 1, NI), lambda b: (b, 0, 0)),
            full((1, H)), full((1, H)), full((H, D)), full((1, D)),
            full((1, H)), full((1, H)), full((H, D)), full((1, D)),
            full((H, D)), full((1, D)), full((D, 1)), full((1, 1)),
            pl.BlockSpec(memory_space=pltpu.SMEM),
            pl.BlockSpec(memory_space=pltpu.SMEM),
        ],
        out_specs=pl.BlockSpec((BS, 1, NUM_QUERY), lambda b: (b, 0, 0)),
    )
    return pl.pallas_call(
        _select_body,
        grid_spec=grid_spec,
        out_shape=jax.ShapeDtypeStruct((B, 1, NUM_QUERY), jnp.int32),
    )(image_features, text_features, gumbel,
      ln_t_scale.reshape(1, H), ln_t_bias.reshape(1, H), W_t, b_t.reshape(1, D),
      ln_i_scale.reshape(1, H), ln_i_bias.reshape(1, H), W_i, b_i.reshape(1, D),
      W_tw1, b_tw1.reshape(1, D), W_tw2, b_tw2.reshape(1, 1),
      temperature.reshape(1, 1), diversity_lambda.reshape(1, 1))


def kernel(image_features, text_features, ln_t_scale, ln_t_bias, W_t, b_t,
           ln_i_scale, ln_i_bias, W_i, b_i, W_tw1, b_tw1, W_tw2, b_tw2,
           temperature, diversity_lambda):
    gum = jax.random.gumbel(jax.random.key(42), (B, NI), jnp.float32)
    sel = _run(image_features, text_features, gum.reshape(B, 1, NI),
               ln_t_scale, ln_t_bias, W_t, b_t, ln_i_scale, ln_i_bias, W_i,
               b_i, W_tw1, b_tw1, W_tw2, b_tw2,
               jnp.asarray(temperature, jnp.float32),
               jnp.asarray(diversity_lambda, jnp.float32))
    return sel.reshape(B, NUM_QUERY).astype(jnp.int64)


# speedup vs baseline: 2.1021x; 1.7794x over previous
import functools

import jax
import jax.numpy as jnp
from jax import lax
from jax.experimental import pallas as pl
from jax.experimental.pallas import tpu as pltpu

B, NI, NT, H, D = 16, 1024, 77, 768, 64
NTP = 80
NUM_QUERY = 16
_SQRT_HALF = 0.7071067811865476


def _gelu(x):
    return 0.5 * x * (1.0 + lax.erf(x * _SQRT_HALF))


def _ln(x, scale, bias):
    m = x.mean(-1, keepdims=True)
    v = ((x - m) ** 2).mean(-1, keepdims=True)
    inv = 1.0 / jnp.sqrt(v + 1e-5)
    return (x - m) * inv * scale + bias


def _select_body(img_ref, txt_ref, gum_ref,
                 ln_t_scale, ln_t_bias, W_t, b_t,
                 ln_i_scale, ln_i_bias, W_i, b_i,
                 W_tw1, b_tw1, W_tw2, b_tw2,
                 temp_ref, lam_ref, sel_ref,
                 logits_scr, ls_scr, idx_scr):
    b = pl.program_id(0)
    lin = lax.broadcasted_iota(jnp.int32, (1, NI), 1)
    lane16 = lax.broadcasted_iota(jnp.int32, (1, NUM_QUERY), 1)

    def argmax_first(v):
        m = jnp.max(v)
        return jnp.min(jnp.where(v == m, lin, NI))

    t = txt_ref[0]
    x = img_ref[0]
    g = gum_ref[0]

    tc = _gelu(_ln(t, ln_t_scale[0], ln_t_bias[0]) @ W_t[...] + b_t[0]) + t[:, :D]
    tw = _gelu(t @ W_tw1[...] + b_tw1[0]) @ W_tw2[...] + b_tw2[0]
    sg = jax.nn.sigmoid(tw)
    sg_m = sg - jnp.max(sg, axis=0, keepdims=True)
    e_t = jnp.exp(sg_m)
    w = e_t / jnp.sum(e_t, axis=0, keepdims=True)
    wt = tc * w
    wt = jnp.concatenate([wt, jnp.zeros((NTP - NT, D), jnp.float32)], axis=0)

    im = _gelu(_ln(x, ln_i_scale[0], ln_i_bias[0]) @ W_i[...] + b_i[0]) + x[:, :D]

    logits = lax.dot_general(wt, im, (((1,), (1,)), ((), ())))
    logits = logits / (jnp.abs(temp_ref[0, 0]) + 1e-6)

    row = lax.broadcasted_iota(jnp.int32, (NTP, 1), 0)
    mx = jnp.max(logits, axis=1, keepdims=True)
    e = jnp.where(row < NT, jnp.exp(logits - mx), 0.0)
    s = jnp.sum(e, axis=1, keepdims=True)
    s = jnp.where(row < NT, s, 1.0)
    sm = e / s
    scores = jnp.max(sm, axis=0, keepdims=True)

    ssum = jnp.sum(scores)
    lp = jnp.log(scores / (ssum + 1e-6))
    ls = jnp.log(scores)

    idx0 = argmax_first(lp + g)

    logits_scr[b] = logits
    ls_scr[b] = ls
    idx_scr[b, 0] = idx0

    @pl.when(b == B - 1)
    def _selection():
        lam = lam_ref[0, 0]
        ones_row = jnp.ones((1, NTP), jnp.float32)
        idxs, Ms, Ss, sels, lss = [], [], [], [], []
        for j in range(B):
            ij = idx_scr[j, 0]
            idxs.append(ij)
            Ms.append(jnp.where(lin == ij, -jnp.inf, jnp.zeros((1, NI), jnp.float32)))
            Ss.append(jnp.zeros((1, NI), jnp.float32))
            sels.append(jnp.full((1, NUM_QUERY), ij, jnp.int32))
            lss.append(ls_scr[j][...])
        for k in range(1, NUM_QUERY):
            for j in range(B):
                lg = logits_scr[j]
                masked = jnp.where(lin == idxs[j], lg, 0.0)
                r = jnp.sum(masked, axis=1, keepdims=True)
                diff2 = (lg - r) ** 2
                d2 = lax.dot_general(ones_row, diff2,
                                     (((1,), (0,)), ((), ())))
                Ss[j] = Ss[j] + jnp.sqrt(d2)
                comb = lss[j] + lam * (Ss[j] / float(k)) + Ms[j]
                idxs[j] = argmax_first(comb)
                sels[j] = jnp.where(lane16 == k, idxs[j], sels[j])
                Ms[j] = jnp.where(lin == idxs[j], -jnp.inf, Ms[j])
        for j in range(B):
            sel_ref[j] = sels[j]


@functools.partial(jax.jit, static_argnames=())
def _run(image_features, text_features, gumbel,
         ln_t_scale, ln_t_bias, W_t, b_t, ln_i_scale, ln_i_bias, W_i, b_i,
         W_tw1, b_tw1, W_tw2, b_tw2, temperature, diversity_lambda):
    full = lambda shape: pl.BlockSpec(shape, lambda b: (0,) * len(shape))
    grid_spec = pl.GridSpec(
        grid=(B,),
        in_specs=[
            pl.BlockSpec((1, NI, H), lambda b: (b, 0, 0)),
            pl.BlockSpec((1, NT, H), lambda b: (b, 0, 0)),
            pl.BlockSpec((1, 1, NI), lambda b: (b, 0, 0)),
            full((1, H)), full((1, H)), full((H, D)), full((1, D)),
            full((1, H)), full((1, H)), full((H, D)), full((1, D)),
            full((H, D)), full((1, D)), full((D, 1)), full((1, 1)),
            pl.BlockSpec(memory_space=pltpu.SMEM),
            pl.BlockSpec(memory_space=pltpu.SMEM),
        ],
        out_specs=pl.BlockSpec((B, 1, NUM_QUERY), lambda b: (0, 0, 0)),
        scratch_shapes=[
            pltpu.VMEM((B, NTP, NI), jnp.float32),
            pltpu.VMEM((B, 1, NI), jnp.float32),
            pltpu.SMEM((B, 1), jnp.int32),
        ],
    )
    return pl.pallas_call(
        _select_body,
        grid_spec=grid_spec,
        out_shape=jax.ShapeDtypeStruct((B, 1, NUM_QUERY), jnp.int32),
    )(image_features, text_features, gumbel,
      ln_t_scale.reshape(1, H), ln_t_bias.reshape(1, H), W_t, b_t.reshape(1, D),
      ln_i_scale.reshape(1, H), ln_i_bias.reshape(1, H), W_i, b_i.reshape(1, D),
      W_tw1, b_tw1.reshape(1, D), W_tw2, b_tw2.reshape(1, 1),
      temperature.reshape(1, 1), diversity_lambda.reshape(1, 1))


def kernel(image_features, text_features, ln_t_scale, ln_t_bias, W_t, b_t,
           ln_i_scale, ln_i_bias, W_i, b_i, W_tw1, b_tw1, W_tw2, b_tw2,
           temperature, diversity_lambda):
    gum = jax.random.gumbel(jax.random.key(42), (B, NI), jnp.float32)
    sel = _run(image_features, text_features, gum.reshape(B, 1, NI),
               ln_t_scale, ln_t_bias, W_t, b_t, ln_i_scale, ln_i_bias, W_i,
               b_i, W_tw1, b_tw1, W_tw2, b_tw2,
               jnp.asarray(temperature, jnp.float32),
               jnp.asarray(diversity_lambda, jnp.float32))
    return sel.reshape(B, NUM_QUERY).astype(jnp.int64)
